# Initial kernel scaffold; baseline (speedup 1.0000x reference)
import jax
import jax.numpy as jnp
from jax.experimental import pallas as pl


def kernel(feature, xyz, neighbour_index, w_mlp1, b_mlp1, g_mlp1, be_mlp1, w_bb1, b_bb1, g_bb1, be_bb1, w_ap1_fc, w_ap1_mlp, b_ap1, g_ap1, be_ap1, w_bb2, b_bb2, g_bb2, be_bb2, w_ap2_fc, w_ap2_mlp, b_ap2, g_ap2, be_ap2, w_mlp2, b_mlp2, g_mlp2, be_mlp2, w_mlp3, b_mlp3, g_mlp3, be_mlp3):
    raise NotImplementedError("write your pallas kernel here")


# speedup vs baseline: 2.8394x; 2.8394x over previous
import functools

import jax
import jax.numpy as jnp
from jax import lax
from jax.experimental import pallas as pl
from jax.experimental.pallas import tpu as pltpu
from jax.experimental.pallas import tpu_sc as plsc

N = 100000
K = 16
M = N * K
NW = 32
CHUNK_J = 16
ROWS_PER_DMA = 128
CHUNK = CHUNK_J * ROWS_PER_DMA
IT = 25
MW = IT * CHUNK
MPAD = NW * MW

BN = 1000
R = BN * K
GRID = N // BN

_LEAK = 0.2


def _leaky(x):
  return jnp.where(x >= 0, x, _LEAK * x)


@functools.lru_cache(maxsize=None)
def _make_sc_gather(d):
  mesh = plsc.VectorSubcoreMesh(
      core_axis_name="c", subcore_axis_name="s", num_cores=2, num_subcores=16)

  @functools.partial(
      pl.kernel,
      mesh=mesh,
      out_type=jax.ShapeDtypeStruct((MPAD, d), jnp.float32),
      scratch_types=[
          pltpu.VMEM((CHUNK_J, ROWS_PER_DMA), jnp.int32),
          pltpu.VMEM((CHUNK, d), jnp.float32),
          pltpu.SemaphoreType.DMA,
      ],
      compiler_params=pltpu.CompilerParams(use_tc_tiling_on_sc=False),
  )
  def gather(table_hbm, idx_hbm, out_hbm, idx_v, rows_v, sem):
    wid = lax.axis_index("s") * 2 + lax.axis_index("c")

    def body(it, _):
      base = (wid * IT + it) * CHUNK
      pltpu.sync_copy(idx_hbm.at[wid, it], idx_v)
      copies = []
      for j in range(CHUNK_J):
        copies.append(
            pltpu.async_copy(
                table_hbm.at[idx_v.at[j]],
                rows_v.at[pl.ds(j * ROWS_PER_DMA, ROWS_PER_DMA)],
                sem,
            )
        )
      for c in copies:
        c.wait()
      pltpu.sync_copy(rows_v, out_hbm.at[pl.ds(base, CHUNK)])
      return _

    lax.fori_loop(0, IT, body, None)

  return gather


def _gather32(table, idx_pad):
  return _make_sc_gather(32)(table, idx_pad)


def _gather16(table, idx_pad):
  return _make_sc_gather(16)(table, idx_pad)


def _pass_a_body(feat, xyz, w1, b1, tab):
  f = _leaky(jnp.dot(feat[...], w1[...],
                     preferred_element_type=jnp.float32) + b1[...])
  tab[:, 0:3] = xyz[...]
  tab[:, 3:19] = f
  tab[:, 19:32] = jnp.zeros((BN, 13), jnp.float32)


def _pass_a(feat, xyz3, w1e, b1e):
  return pl.pallas_call(
      _pass_a_body,
      grid=(GRID,),
      in_specs=[
          pl.BlockSpec((BN, 8), lambda i: (i, 0)),
          pl.BlockSpec((BN, 3), lambda i: (i, 0)),
          pl.BlockSpec((8, 16), lambda i: (0, 0)),
          pl.BlockSpec((1, 16), lambda i: (0, 0)),
      ],
      out_specs=pl.BlockSpec((BN, 32), lambda i: (i, 0)),
      out_shape=jax.ShapeDtypeStruct((N, 32), jnp.float32),
  )(feat, xyz3, w1e, b1e)


def _pass_c_body(gth, tab, wbb1, bbb1, wfc1, wap1, bap1, wbb2, bbb2,
                 agg1_o, fx2_o):
  g = gth[...]
  neigh_xyz = g[:, 0:3]
  f_neigh = g[:, 3:19]
  tile3 = tab[:, 0:3]
  tile_r = jnp.broadcast_to(tile3[:, None, :], (BN, K, 3)).reshape(R, 3)
  rel = tile_r - neigh_xyz
  dist = jnp.sqrt(jnp.sum(rel * rel, axis=1, keepdims=True) + 1e-12)
  fx = jnp.concatenate([dist, rel, tile_r, neigh_xyz], axis=1)
  fx1 = _leaky(jnp.dot(fx, wbb1[...],
                       preferred_element_type=jnp.float32) + bbb1[...])
  fc1 = jnp.concatenate([f_neigh, fx1], axis=1)
  t = jnp.dot(fc1, wfc1[...], preferred_element_type=jnp.float32)
  t3 = t.reshape(BN, K, 32)
  m = jnp.max(t3, axis=1, keepdims=True)
  e = jnp.exp(t3 - m)
  score = e / jnp.sum(e, axis=1, keepdims=True)
  s = jnp.sum(t3 * score, axis=1)
  agg1_o[...] = _leaky(jnp.dot(s, wap1[...],
                               preferred_element_type=jnp.float32) + bap1[...])
  fx2_o[...] = _leaky(jnp.dot(fx1, wbb2[...],
                              preferred_element_type=jnp.float32) + bbb2[...])


def _pass_c(gth, tab, wbb1e, bbb1e, wfc1t, wap1e, bap1e, wbb2e, bbb2e):
  return pl.pallas_call(
      _pass_c_body,
      grid=(GRID,),
      in_specs=[
          pl.BlockSpec((R, 32), lambda i: (i, 0)),
          pl.BlockSpec((BN, 32), lambda i: (i, 0)),
          pl.BlockSpec((10, 16), lambda i: (0, 0)),
          pl.BlockSpec((1, 16), lambda i: (0, 0)),
          pl.BlockSpec((32, 32), lambda i: (0, 0)),
          pl.BlockSpec((32, 16), lambda i: (0, 0)),
          pl.BlockSpec((1, 16), lambda i: (0, 0)),
          pl.BlockSpec((16, 16), lambda i: (0, 0)),
          pl.BlockSpec((1, 16), lambda i: (0, 0)),
      ],
      out_specs=[
          pl.BlockSpec((BN, 16), lambda i: (i, 0)),
          pl.BlockSpec((R, 16), lambda i: (i, 0)),
      ],
      out_shape=[
          jax.ShapeDtypeStruct((N, 16), jnp.float32),
          jax.ShapeDtypeStruct((M, 16), jnp.float32),
      ],
  )(gth, tab, wbb1e, bbb1e, wfc1t, wap1e, bap1e, wbb2e, bbb2e)


def _pass_e_body(gth2, fx2, feat, wfc2, wap2, bap2, wm2, bm2, wm3, bm3, out):
  fc2 = jnp.concatenate([gth2[...], fx2[...]], axis=1)
  t = jnp.dot(fc2, wfc2[...], preferred_element_type=jnp.float32)
  t3 = t.reshape(BN, K, 32)
  m = jnp.max(t3, axis=1, keepdims=True)
  e = jnp.exp(t3 - m)
  score = e / jnp.sum(e, axis=1, keepdims=True)
  s = jnp.sum(t3 * score, axis=1)
  agg2 = _leaky(jnp.dot(s, wap2[...],
                        preferred_element_type=jnp.float32) + bap2[...])
  fp2 = jnp.dot(agg2, wm2[...], preferred_element_type=jnp.float32) + bm2[...]
  scp = jnp.dot(feat[...], wm3[...],
                preferred_element_type=jnp.float32) + bm3[...]
  out[...] = _leaky(fp2 + scp)


def _pass_e(gth2, fx2, feat, wfc2t, wap2e, bap2e, wm2e, bm2e, wm3e, bm3e):
  return pl.pallas_call(
      _pass_e_body,
      grid=(GRID,),
      in_specs=[
          pl.BlockSpec((R, 16), lambda i: (i, 0)),
          pl.BlockSpec((R, 16), lambda i: (i, 0)),
          pl.BlockSpec((BN, 8), lambda i: (i, 0)),
          pl.BlockSpec((32, 32), lambda i: (0, 0)),
          pl.BlockSpec((32, 32), lambda i: (0, 0)),
          pl.BlockSpec((1, 32), lambda i: (0, 0)),
          pl.BlockSpec((32, 64), lambda i: (0, 0)),
          pl.BlockSpec((1, 64), lambda i: (0, 0)),
          pl.BlockSpec((8, 64), lambda i: (0, 0)),
          pl.BlockSpec((1, 64), lambda i: (0, 0)),
      ],
      out_specs=pl.BlockSpec((BN, 64), lambda i: (i, 0)),
      out_shape=jax.ShapeDtypeStruct((N, 64), jnp.float32),
  )(gth2, fx2, feat, wfc2t, wap2e, bap2e, wm2e, bm2e, wm3e, bm3e)


def _eff(w, b, g, be):
  we = (g[:, None] * w).T
  be_ = (g * b + be).reshape(1, -1)
  return we.astype(jnp.float32), be_.astype(jnp.float32)


def kernel(feature, xyz, neighbour_index,
           w_mlp1, b_mlp1, g_mlp1, be_mlp1,
           w_bb1, b_bb1, g_bb1, be_bb1,
           w_ap1_fc,
           w_ap1_mlp, b_ap1, g_ap1, be_ap1,
           w_bb2, b_bb2, g_bb2, be_bb2,
           w_ap2_fc,
           w_ap2_mlp, b_ap2, g_ap2, be_ap2,
           w_mlp2, b_mlp2, g_mlp2, be_mlp2,
           w_mlp3, b_mlp3, g_mlp3, be_mlp3):
  feat = feature[0, :, :, 0].T
  xyz3 = xyz[0]

  w1e, b1e = _eff(w_mlp1, b_mlp1, g_mlp1, be_mlp1)
  wbb1e, bbb1e = _eff(w_bb1, b_bb1, g_bb1, be_bb1)
  wap1e, bap1e = _eff(w_ap1_mlp, b_ap1, g_ap1, be_ap1)
  wbb2e, bbb2e = _eff(w_bb2, b_bb2, g_bb2, be_bb2)
  wap2e, bap2e = _eff(w_ap2_mlp, b_ap2, g_ap2, be_ap2)
  wm2e, bm2e = _eff(w_mlp2, b_mlp2, g_mlp2, be_mlp2)
  wm3e, bm3e = _eff(w_mlp3, b_mlp3, g_mlp3, be_mlp3)
  wfc1t = w_ap1_fc.T
  wfc2t = w_ap2_fc.T

  idx_flat = neighbour_index.reshape(-1)
  idx_pad = jnp.concatenate(
      [idx_flat, jnp.zeros((MPAD - M,), jnp.int32)]
  ).reshape(NW, IT, CHUNK_J, ROWS_PER_DMA)

  table = _pass_a(feat, xyz3, w1e, b1e)
  gth = _gather32(table, idx_pad)
  agg1, fx2 = _pass_c(gth, table, wbb1e, bbb1e, wfc1t,
                      wap1e, bap1e, wbb2e, bbb2e)
  gth2 = _gather16(agg1, idx_pad)
  out = _pass_e(gth2, fx2, feat, wfc2t, wap2e, bap2e,
                wm2e, bm2e, wm3e, bm3e)
  return out.T.reshape(1, 2 * 32, N, 1)


# speedup vs baseline: 3.1822x; 1.1207x over previous
import functools

import jax
import jax.numpy as jnp
from jax import lax
from jax.experimental import pallas as pl
from jax.experimental.pallas import tpu as pltpu
from jax.experimental.pallas import tpu_sc as plsc

N = 100000
K = 16
NP = 102400
NW = 32
CHUNK_J = 8
ROWS_PER_DMA = 128
CHUNK = CHUNK_J * ROWS_PER_DMA
IT = 50
ITH = IT // 2
MW = IT * CHUNK
MPAD = NW * MW

BN = 1024
R = BN * K
GRID = -(-N // BN)

_LEAK = 0.2


def _leaky(x):
  return jnp.where(x >= 0, x, _LEAK * x)


@functools.lru_cache(maxsize=None)
def _make_sc_gather(d):
  mesh = plsc.VectorSubcoreMesh(
      core_axis_name="c", subcore_axis_name="s", num_cores=2, num_subcores=16)

  @functools.partial(
      pl.kernel,
      mesh=mesh,
      out_type=jax.ShapeDtypeStruct((MPAD, d), jnp.float32),
      scratch_types=[
          pltpu.VMEM((CHUNK_J, ROWS_PER_DMA), jnp.int32),
          pltpu.VMEM((CHUNK_J, ROWS_PER_DMA), jnp.int32),
          pltpu.VMEM((CHUNK, d), jnp.float32),
          pltpu.VMEM((CHUNK, d), jnp.float32),
          pltpu.SemaphoreType.DMA,
          pltpu.SemaphoreType.DMA,
          pltpu.SemaphoreType.DMA,
      ],
      compiler_params=pltpu.CompilerParams(use_tc_tiling_on_sc=False),
  )
  def gather(table_hbm, idx_hbm, out_hbm, idx0, idx1, rows0, rows1,
             semg, semw0, semw1):
    wid = lax.axis_index("s") * 2 + lax.axis_index("c")

    def one_chunk(chunk, idx_v, rows_v, semw):
      base = (wid * IT + chunk) * CHUNK
      pltpu.sync_copy(idx_hbm.at[wid, chunk], idx_v)
      copies = []
      for j in range(CHUNK_J):
        copies.append(
            pltpu.async_copy(
                table_hbm.at[idx_v.at[j]],
                rows_v.at[pl.ds(j * ROWS_PER_DMA, ROWS_PER_DMA)],
                semg,
            )
        )
      for c in copies:
        c.wait()
      pltpu.async_copy(rows_v, out_hbm.at[pl.ds(base, CHUNK)], semw)

    def body(j, _):
      @pl.when(j >= 1)
      def _drain0():
        pltpu.make_async_copy(
            out_hbm.at[pl.ds(0, CHUNK)], rows0, semw0).wait()

      one_chunk(2 * j, idx0, rows0, semw0)

      @pl.when(j >= 1)
      def _drain1():
        pltpu.make_async_copy(
            out_hbm.at[pl.ds(0, CHUNK)], rows1, semw1).wait()

      one_chunk(2 * j + 1, idx1, rows1, semw1)
      return _

    lax.fori_loop(0, ITH, body, None)
    pltpu.make_async_copy(out_hbm.at[pl.ds(0, CHUNK)], rows0, semw0).wait()
    pltpu.make_async_copy(out_hbm.at[pl.ds(0, CHUNK)], rows1, semw1).wait()

  return gather


def _gather32(table, idx_pad):
  return _make_sc_gather(32)(table, idx_pad)


def _gather16(table, idx_pad):
  return _make_sc_gather(16)(table, idx_pad)


def _pass_a_body(feat, xyz, w1, b1, tab):
  f = _leaky(jnp.dot(feat[...], w1[...],
                     preferred_element_type=jnp.float32) + b1[...])
  tab[:, 0:3] = xyz[...]
  tab[:, 3:19] = f
  tab[:, 19:32] = jnp.zeros((tab.shape[0], 13), jnp.float32)


def _pass_a(feat, xyz3, w1e, b1e):
  return pl.pallas_call(
      _pass_a_body,
      grid=(GRID,),
      in_specs=[
          pl.BlockSpec((BN, 8), lambda i: (i, 0)),
          pl.BlockSpec((BN, 3), lambda i: (i, 0)),
          pl.BlockSpec((8, 16), lambda i: (0, 0)),
          pl.BlockSpec((1, 16), lambda i: (0, 0)),
      ],
      out_specs=pl.BlockSpec((BN, 32), lambda i: (i, 0)),
      out_shape=jax.ShapeDtypeStruct((N, 32), jnp.float32),
  )(feat, xyz3, w1e, b1e)


def _pass_c_body(gth, tab, wbb1, bbb1, wfc1, wap1, bap1, wbb2, bbb2,
                 agg1_o, fx2_o):
  g3 = gth[...]
  neigh_xyz = g3[:, :, 0:3]
  tile3 = tab[:, 0:3][None]
  tile_b = jnp.broadcast_to(tile3, (K, BN, 3))
  rel = tile_b - neigh_xyz
  dist = jnp.sqrt(jnp.sum(rel * rel, axis=2, keepdims=True) + 1e-12)
  fx = jnp.concatenate([dist, rel, tile_b, neigh_xyz], axis=2)
  fx1 = _leaky(jnp.dot(fx.reshape(R, 10), wbb1[...],
                       preferred_element_type=jnp.float32) + bbb1[...])
  f_neigh = g3[:, :, 3:19].reshape(R, 16)
  fc1 = jnp.concatenate([f_neigh, fx1], axis=1)
  t = jnp.dot(fc1, wfc1[...], preferred_element_type=jnp.float32)
  t3 = t.reshape(K, BN, 32)
  m = jnp.max(t3, axis=0, keepdims=True)
  e = jnp.exp(t3 - m)
  s = jnp.sum(t3 * e, axis=0) / jnp.sum(e, axis=0)
  agg1_o[...] = _leaky(jnp.dot(s, wap1[...],
                               preferred_element_type=jnp.float32) + bap1[...])
  fx2 = _leaky(jnp.dot(fx1, wbb2[...],
                       preferred_element_type=jnp.float32) + bbb2[...])
  fx2_o[...] = fx2.reshape(K, BN, 16)


def _pass_c(gth3, tab, wbb1e, bbb1e, wfc1t, wap1e, bap1e, wbb2e, bbb2e):
  return pl.pallas_call(
      _pass_c_body,
      grid=(GRID,),
      in_specs=[
          pl.BlockSpec((K, BN, 32), lambda i: (0, i, 0)),
          pl.BlockSpec((BN, 32), lambda i: (i, 0)),
          pl.BlockSpec((10, 16), lambda i: (0, 0)),
          pl.BlockSpec((1, 16), lambda i: (0, 0)),
          pl.BlockSpec((32, 32), lambda i: (0, 0)),
          pl.BlockSpec((32, 16), lambda i: (0, 0)),
          pl.BlockSpec((1, 16), lambda i: (0, 0)),
          pl.BlockSpec((16, 16), lambda i: (0, 0)),
          pl.BlockSpec((1, 16), lambda i: (0, 0)),
      ],
      out_specs=[
          pl.BlockSpec((BN, 16), lambda i: (i, 0)),
          pl.BlockSpec((K, BN, 16), lambda i: (0, i, 0)),
      ],
      out_shape=[
          jax.ShapeDtypeStruct((N, 16), jnp.float32),
          jax.ShapeDtypeStruct((K, N, 16), jnp.float32),
      ],
  )(gth3, tab, wbb1e, bbb1e, wfc1t, wap1e, bap1e, wbb2e, bbb2e)


def _pass_e_body(gth2, fx2, feat, wfc2, wap2, bap2, wm2, bm2, wm3, bm3, out):
  fc2 = jnp.concatenate([gth2[...], fx2[...]], axis=2)
  t = jnp.dot(fc2.reshape(R, 32), wfc2[...],
              preferred_element_type=jnp.float32)
  t3 = t.reshape(K, BN, 32)
  m = jnp.max(t3, axis=0, keepdims=True)
  e = jnp.exp(t3 - m)
  s = jnp.sum(t3 * e, axis=0) / jnp.sum(e, axis=0)
  agg2 = _leaky(jnp.dot(s, wap2[...],
                        preferred_element_type=jnp.float32) + bap2[...])
  fp2 = jnp.dot(agg2, wm2[...], preferred_element_type=jnp.float32) + bm2[...]
  scp = jnp.dot(feat[...], wm3[...],
                preferred_element_type=jnp.float32) + bm3[...]
  out[...] = _leaky(fp2 + scp).T


def _pass_e(gth2, fx2, feat, wfc2t, wap2e, bap2e, wm2e, bm2e, wm3e, bm3e):
  return pl.pallas_call(
      _pass_e_body,
      grid=(GRID,),
      in_specs=[
          pl.BlockSpec((K, BN, 16), lambda i: (0, i, 0)),
          pl.BlockSpec((K, BN, 16), lambda i: (0, i, 0)),
          pl.BlockSpec((BN, 8), lambda i: (i, 0)),
          pl.BlockSpec((32, 32), lambda i: (0, 0)),
          pl.BlockSpec((32, 32), lambda i: (0, 0)),
          pl.BlockSpec((1, 32), lambda i: (0, 0)),
          pl.BlockSpec((32, 64), lambda i: (0, 0)),
          pl.BlockSpec((1, 64), lambda i: (0, 0)),
          pl.BlockSpec((8, 64), lambda i: (0, 0)),
          pl.BlockSpec((1, 64), lambda i: (0, 0)),
      ],
      out_specs=pl.BlockSpec((64, BN), lambda i: (0, i)),
      out_shape=jax.ShapeDtypeStruct((64, N), jnp.float32),
  )(gth2, fx2, feat, wfc2t, wap2e, bap2e, wm2e, bm2e, wm3e, bm3e)


def _eff(w, b, g, be):
  we = (g[:, None] * w).T
  be_ = (g * b + be).reshape(1, -1)
  return we.astype(jnp.float32), be_.astype(jnp.float32)


def kernel(feature, xyz, neighbour_index,
           w_mlp1, b_mlp1, g_mlp1, be_mlp1,
           w_bb1, b_bb1, g_bb1, be_bb1,
           w_ap1_fc,
           w_ap1_mlp, b_ap1, g_ap1, be_ap1,
           w_bb2, b_bb2, g_bb2, be_bb2,
           w_ap2_fc,
           w_ap2_mlp, b_ap2, g_ap2, be_ap2,
           w_mlp2, b_mlp2, g_mlp2, be_mlp2,
           w_mlp3, b_mlp3, g_mlp3, be_mlp3):
  feat = feature[0, :, :, 0].T
  xyz3 = xyz[0]

  w1e, b1e = _eff(w_mlp1, b_mlp1, g_mlp1, be_mlp1)
  wbb1e, bbb1e = _eff(w_bb1, b_bb1, g_bb1, be_bb1)
  wap1e, bap1e = _eff(w_ap1_mlp, b_ap1, g_ap1, be_ap1)
  wbb2e, bbb2e = _eff(w_bb2, b_bb2, g_bb2, be_bb2)
  wap2e, bap2e = _eff(w_ap2_mlp, b_ap2, g_ap2, be_ap2)
  wm2e, bm2e = _eff(w_mlp2, b_mlp2, g_mlp2, be_mlp2)
  wm3e, bm3e = _eff(w_mlp3, b_mlp3, g_mlp3, be_mlp3)
  wfc1t = w_ap1_fc.T
  wfc2t = w_ap2_fc.T

  idxt = jnp.pad(neighbour_index[0].T, ((0, 0), (0, NP - N)))
  idx_pad = idxt.reshape(NW, IT, CHUNK_J, ROWS_PER_DMA)

  table = _pass_a(feat, xyz3, w1e, b1e)
  gth = _gather32(table, idx_pad).reshape(K, NP, 32)
  agg1, fx2 = _pass_c(gth, table, wbb1e, bbb1e, wfc1t,
                      wap1e, bap1e, wbb2e, bbb2e)
  gth2 = _gather16(agg1, idx_pad).reshape(K, NP, 16)
  out = _pass_e(gth2, fx2, feat, wfc2t, wap2e, bap2e,
                wm2e, bm2e, wm3e, bm3e)
  return out.reshape(1, 2 * 32, N, 1)


# speedup vs baseline: 6.8164x; 2.1420x over previous
import functools

import jax
import jax.numpy as jnp
from jax import lax
from jax.experimental import pallas as pl
from jax.experimental.pallas import tpu as pltpu
from jax.experimental.pallas import tpu_sc as plsc

N = 100000
K = 16
NP = 102400
NW = 32
CHUNK_J = 8
ROWS_PER_DMA = 128
CHUNK = CHUNK_J * ROWS_PER_DMA
IT = 50
ITH = IT // 2
MW = IT * CHUNK
MPAD = NW * MW

BN = 1024
R = BN * K
GRID = -(-N // BN)
B4 = BN // 4
R4 = R // 4
NP4 = NP // 4

_LEAK = 0.2


def _leaky(x):
  return jnp.where(x >= 0, x, _LEAK * x)


@functools.lru_cache(maxsize=None)
def _make_sc_gather(d):
  mesh = plsc.VectorSubcoreMesh(
      core_axis_name="c", subcore_axis_name="s", num_cores=2, num_subcores=16)

  @functools.partial(
      pl.kernel,
      mesh=mesh,
      out_type=jax.ShapeDtypeStruct((MPAD, d), jnp.float32),
      scratch_types=[
          pltpu.VMEM((CHUNK_J, ROWS_PER_DMA), jnp.int32),
          pltpu.VMEM((CHUNK_J, ROWS_PER_DMA), jnp.int32),
          pltpu.VMEM((CHUNK, d), jnp.float32),
          pltpu.VMEM((CHUNK, d), jnp.float32),
          pltpu.SemaphoreType.DMA,
          pltpu.SemaphoreType.DMA,
          pltpu.SemaphoreType.DMA,
      ],
      compiler_params=pltpu.CompilerParams(use_tc_tiling_on_sc=False),
  )
  def gather(table_hbm, idx_hbm, out_hbm, idx0, idx1, rows0, rows1,
             semg, semw0, semw1):
    wid = lax.axis_index("s") * 2 + lax.axis_index("c")

    def one_chunk(chunk, idx_v, rows_v, semw):
      base = (wid * IT + chunk) * CHUNK
      pltpu.sync_copy(idx_hbm.at[wid, chunk], idx_v)
      copies = []
      for j in range(CHUNK_J):
        copies.append(
            pltpu.async_copy(
                table_hbm.at[idx_v.at[j]],
                rows_v.at[pl.ds(j * ROWS_PER_DMA, ROWS_PER_DMA)],
                semg,
            )
        )
      for c in copies:
        c.wait()
      pltpu.async_copy(rows_v, out_hbm.at[pl.ds(base, CHUNK)], semw)

    def body(j, _):
      @pl.when(j >= 1)
      def _drain0():
        pltpu.make_async_copy(
            out_hbm.at[pl.ds(0, CHUNK)], rows0, semw0).wait()

      one_chunk(2 * j, idx0, rows0, semw0)

      @pl.when(j >= 1)
      def _drain1():
        pltpu.make_async_copy(
            out_hbm.at[pl.ds(0, CHUNK)], rows1, semw1).wait()

      one_chunk(2 * j + 1, idx1, rows1, semw1)
      return _

    lax.fori_loop(0, ITH, body, None)
    pltpu.make_async_copy(out_hbm.at[pl.ds(0, CHUNK)], rows0, semw0).wait()
    pltpu.make_async_copy(out_hbm.at[pl.ds(0, CHUNK)], rows1, semw1).wait()

  return gather


def _gather32(table, idx_pad):
  return _make_sc_gather(32)(table, idx_pad)


def _gather16(table, idx_pad):
  return _make_sc_gather(16)(table, idx_pad)


def _pass_a_body(feat, xyz, w1, b1, tab):
  f = _leaky(jnp.dot(feat[...], w1[...],
                     preferred_element_type=jnp.float32) + b1[...])
  tab[:, 0:3] = xyz[...]
  tab[:, 3:19] = f
  tab[:, 19:32] = jnp.zeros((tab.shape[0], 13), jnp.float32)


def _pass_a(feat, xyz3, w1e, b1e):
  return pl.pallas_call(
      _pass_a_body,
      grid=(GRID,),
      in_specs=[
          pl.BlockSpec((BN, 8), lambda i: (i, 0)),
          pl.BlockSpec((BN, 3), lambda i: (i, 0)),
          pl.BlockSpec((8, 16), lambda i: (0, 0)),
          pl.BlockSpec((1, 16), lambda i: (0, 0)),
      ],
      out_specs=pl.BlockSpec((BN, 32), lambda i: (i, 0)),
      out_shape=jax.ShapeDtypeStruct((N, 32), jnp.float32),
  )(feat, xyz3, w1e, b1e)


def _pass_c_body(gth, tab, s4w, d4w, n4w, t4w, f4w, x4w, b1v, a4w, bap1v,
                 bb4w, bbb2v, tab2_o, fx2_o):
  g3 = gth[...]
  g2 = g3.reshape(R4, 128)
  tp = tab[...]
  rel = tp[None] - g3
  rp = (rel * rel).reshape(R4, 128)
  d2 = jnp.dot(rp, s4w[...], preferred_element_type=jnp.float32)
  dv = jnp.sqrt(d2 + 1e-12)
  tt = jnp.dot(tp, t4w[...], preferred_element_type=jnp.float32)
  fx1 = _leaky(
      (jnp.dot(dv, d4w[...], preferred_element_type=jnp.float32)
       + jnp.dot(g2, n4w[...], preferred_element_type=jnp.float32)
       ).reshape(K, B4, 128) + tt[None] + b1v[...][None]).reshape(R4, 128)
  t = (jnp.dot(g2, f4w[...], preferred_element_type=jnp.float32)
       + jnp.dot(fx1, x4w[...], preferred_element_type=jnp.float32))
  t3 = t.reshape(K, B4, 128)
  m = jnp.max(t3, axis=0, keepdims=True)
  e = jnp.exp(t3 - m)
  s = jnp.sum(t3 * e, axis=0) / jnp.sum(e, axis=0)
  tab2_o[...] = _leaky(jnp.dot(s, a4w[...],
                               preferred_element_type=jnp.float32)
                       + bap1v[...])
  fx2 = _leaky(jnp.dot(fx1, bb4w[...], preferred_element_type=jnp.float32)
               + bbb2v[...][None])
  fx2_o[...] = fx2.reshape(K, B4, 128)


def _pass_c(gth3, tabp, s4w, d4w, n4w, t4w, f4w, x4w, b1v, a4w, bap1v,
            bb4w, bbb2v):
  wspec = pl.BlockSpec((128, 128), lambda i: (0, 0))
  vspec = pl.BlockSpec((1, 128), lambda i: (0, 0))
  return pl.pallas_call(
      _pass_c_body,
      grid=(GRID,),
      in_specs=[
          pl.BlockSpec((K, B4, 128), lambda i: (0, i, 0)),
          pl.BlockSpec((B4, 128), lambda i: (i, 0)),
          wspec, wspec, wspec, wspec, wspec, wspec, vspec, wspec, vspec,
          wspec, vspec,
      ],
      out_specs=[
          pl.BlockSpec((B4, 128), lambda i: (i, 0)),
          pl.BlockSpec((K, B4, 128), lambda i: (0, i, 0)),
      ],
      out_shape=[
          jax.ShapeDtypeStruct((N // 4, 128), jnp.float32),
          jax.ShapeDtypeStruct((K, NP4, 128), jnp.float32),
      ],
  )(gth3, tabp, s4w, d4w, n4w, t4w, f4w, x4w, b1v, a4w, bap1v, bb4w, bbb2v)


def _pass_e_body(gth2, fx2, l24w, x24w, a24w, b24v, agg2_o):
  g2 = gth2[...].reshape(R4, 128)
  f2 = fx2[...].reshape(R4, 128)
  t = (jnp.dot(g2, l24w[...], preferred_element_type=jnp.float32)
       + jnp.dot(f2, x24w[...], preferred_element_type=jnp.float32))
  t3 = t.reshape(K, B4, 128)
  m = jnp.max(t3, axis=0, keepdims=True)
  e = jnp.exp(t3 - m)
  s = jnp.sum(t3 * e, axis=0) / jnp.sum(e, axis=0)
  agg2_o[...] = _leaky(jnp.dot(s, a24w[...],
                               preferred_element_type=jnp.float32)
                       + b24v[...])


def _pass_e(gth2, fx2, l24w, x24w, a24w, b24v):
  wspec = pl.BlockSpec((128, 128), lambda i: (0, 0))
  vspec = pl.BlockSpec((1, 128), lambda i: (0, 0))
  return pl.pallas_call(
      _pass_e_body,
      grid=(GRID,),
      in_specs=[
          pl.BlockSpec((K, B4, 128), lambda i: (0, i, 0)),
          pl.BlockSpec((K, B4, 128), lambda i: (0, i, 0)),
          wspec, wspec, wspec, vspec,
      ],
      out_specs=pl.BlockSpec((B4, 128), lambda i: (i, 0)),
      out_shape=jax.ShapeDtypeStruct((N // 4, 128), jnp.float32),
  )(gth2, fx2, l24w, x24w, a24w, b24v)


def _pass_f_body(agg2, feat, wm2, bm2, wm3, bm3, out):
  fp2 = jnp.dot(agg2[...], wm2[...],
                preferred_element_type=jnp.float32) + bm2[...]
  scp = jnp.dot(feat[...], wm3[...],
                preferred_element_type=jnp.float32) + bm3[...]
  out[...] = _leaky(fp2 + scp).T


def _pass_f(agg2r, feat, wm2e, bm2e, wm3e, bm3e):
  return pl.pallas_call(
      _pass_f_body,
      grid=(GRID,),
      in_specs=[
          pl.BlockSpec((BN, 32), lambda i: (i, 0)),
          pl.BlockSpec((BN, 8), lambda i: (i, 0)),
          pl.BlockSpec((32, 64), lambda i: (0, 0)),
          pl.BlockSpec((1, 64), lambda i: (0, 0)),
          pl.BlockSpec((8, 64), lambda i: (0, 0)),
          pl.BlockSpec((1, 64), lambda i: (0, 0)),
      ],
      out_specs=pl.BlockSpec((64, BN), lambda i: (0, i)),
      out_shape=jax.ShapeDtypeStruct((64, N), jnp.float32),
  )(agg2r, feat, wm2e, bm2e, wm3e, bm3e)


def _eff(w, b, g, be):
  we = (g[:, None] * w).T
  be_ = (g * b + be).reshape(1, -1)
  return we.astype(jnp.float32), be_.astype(jnp.float32)


def kernel(feature, xyz, neighbour_index,
           w_mlp1, b_mlp1, g_mlp1, be_mlp1,
           w_bb1, b_bb1, g_bb1, be_bb1,
           w_ap1_fc,
           w_ap1_mlp, b_ap1, g_ap1, be_ap1,
           w_bb2, b_bb2, g_bb2, be_bb2,
           w_ap2_fc,
           w_ap2_mlp, b_ap2, g_ap2, be_ap2,
           w_mlp2, b_mlp2, g_mlp2, be_mlp2,
           w_mlp3, b_mlp3, g_mlp3, be_mlp3):
  feat = feature[0, :, :, 0].T
  xyz3 = xyz[0]

  w1e, b1e = _eff(w_mlp1, b_mlp1, g_mlp1, be_mlp1)
  wbb1e, bbb1e = _eff(w_bb1, b_bb1, g_bb1, be_bb1)
  wap1e, bap1e = _eff(w_ap1_mlp, b_ap1, g_ap1, be_ap1)
  wbb2e, bbb2e = _eff(w_bb2, b_bb2, g_bb2, be_bb2)
  wap2e, bap2e = _eff(w_ap2_mlp, b_ap2, g_ap2, be_ap2)
  wm2e, bm2e = _eff(w_mlp2, b_mlp2, g_mlp2, be_mlp2)
  wm3e, bm3e = _eff(w_mlp3, b_mlp3, g_mlp3, be_mlp3)
  wfc1t = w_ap1_fc.T
  wfc2t = w_ap2_fc.T

  idxt = jnp.pad(neighbour_index[0].T, ((0, 0), (0, NP - N)))
  idx_pad = idxt.reshape(NW, IT, CHUNK_J, ROWS_PER_DMA)

  def bd4(w, roff, coff):
    z = jnp.zeros((128, 128), jnp.float32)
    h, wd = w.shape
    for a in range(4):
      z = z.at[32 * a + roff:32 * a + roff + h,
               32 * a + coff:32 * a + coff + wd].set(w)
    return z

  def lane4(v, off=0):
    z = jnp.zeros((32,), jnp.float32).at[off:off + v.shape[0]].set(v)
    return jnp.tile(z, 4)[None]

  s4w = bd4(jnp.ones((3, 1), jnp.float32), 0, 0)
  d4w = bd4(wbb1e[0:1, :], 0, 0)
  n4w = bd4(wbb1e[7:10, :] - wbb1e[1:4, :], 0, 0)
  t4w = bd4(wbb1e[1:4, :] + wbb1e[4:7, :], 0, 0)
  f4w = bd4(wfc1t[0:16, :], 3, 0)
  x4w = bd4(wfc1t[16:32, :], 0, 0)
  b1v = lane4(bbb1e.reshape(-1))
  a4w = bd4(wap1e, 0, 0)
  bap1v = lane4(bap1e.reshape(-1))
  bb4w = bd4(wbb2e, 0, 0)
  bbb2v = lane4(bbb2e.reshape(-1))
  l24w = bd4(wfc2t[0:16, :], 0, 0)
  x24w = bd4(wfc2t[16:32, :], 0, 0)
  a24w = bd4(wap2e, 0, 0)
  b24v = lane4(bap2e.reshape(-1))

  table = _pass_a(feat, xyz3, w1e, b1e)
  tablep = table.reshape(N // 4, 128)
  gth = _gather32(table, idx_pad).reshape(K, NP4, 128)
  tab2p, fx2 = _pass_c(gth, tablep, s4w, d4w, n4w, t4w, f4w, x4w, b1v,
                       a4w, bap1v, bb4w, bbb2v)
  gth2 = _gather32(tab2p.reshape(N, 32), idx_pad).reshape(K, NP4, 128)
  agg2p = _pass_e(gth2, fx2, l24w, x24w, a24w, b24v)
  out = _pass_f(agg2p.reshape(N, 32), feat,
                wm2e, bm2e, wm3e, bm3e)
  return out.reshape(1, 2 * 32, N, 1)


# speedup vs baseline: 6.9386x; 1.0179x over previous
import functools

import jax
import jax.numpy as jnp
from jax import lax
from jax.experimental import pallas as pl
from jax.experimental.pallas import tpu as pltpu
from jax.experimental.pallas import tpu_sc as plsc

N = 100000
K = 16
NP = 102400
NW = 32
CHUNK_J = 10
ROWS_PER_DMA = 128
CHUNK = CHUNK_J * ROWS_PER_DMA
IT = 40
ITH = IT // 2
MW = IT * CHUNK
MPAD = NW * MW

BN = 1024
R = BN * K
GRID = -(-N // BN)
B4 = BN // 4
R4 = R // 4
NP4 = NP // 4

_LEAK = 0.2


def _leaky(x):
  return jnp.where(x >= 0, x, _LEAK * x)


@functools.lru_cache(maxsize=None)
def _make_sc_gather(d):
  mesh = plsc.VectorSubcoreMesh(
      core_axis_name="c", subcore_axis_name="s", num_cores=2, num_subcores=16)

  @functools.partial(
      pl.kernel,
      mesh=mesh,
      out_type=jax.ShapeDtypeStruct((MPAD, d), jnp.float32),
      scratch_types=[
          pltpu.VMEM((CHUNK_J, ROWS_PER_DMA), jnp.int32),
          pltpu.VMEM((CHUNK_J, ROWS_PER_DMA), jnp.int32),
          pltpu.VMEM((CHUNK, d), jnp.float32),
          pltpu.VMEM((CHUNK, d), jnp.float32),
          pltpu.SemaphoreType.DMA,
          pltpu.SemaphoreType.DMA,
          pltpu.SemaphoreType.DMA,
      ],
      compiler_params=pltpu.CompilerParams(use_tc_tiling_on_sc=False),
  )
  def gather(table_hbm, idx_hbm, out_hbm, idx0, idx1, rows0, rows1,
             semg, semw0, semw1):
    wid = lax.axis_index("s") * 2 + lax.axis_index("c")

    def one_chunk(chunk, idx_v, rows_v, semw):
      base = (wid * IT + chunk) * CHUNK
      pltpu.sync_copy(idx_hbm.at[wid, chunk], idx_v)
      copies = []
      for j in range(CHUNK_J):
        copies.append(
            pltpu.async_copy(
                table_hbm.at[idx_v.at[j]],
                rows_v.at[pl.ds(j * ROWS_PER_DMA, ROWS_PER_DMA)],
                semg,
            )
        )
      for c in copies:
        c.wait()
      pltpu.async_copy(rows_v, out_hbm.at[pl.ds(base, CHUNK)], semw)

    def body(j, _):
      @pl.when(j >= 1)
      def _drain0():
        pltpu.make_async_copy(
            out_hbm.at[pl.ds(0, CHUNK)], rows0, semw0).wait()

      one_chunk(2 * j, idx0, rows0, semw0)

      @pl.when(j >= 1)
      def _drain1():
        pltpu.make_async_copy(
            out_hbm.at[pl.ds(0, CHUNK)], rows1, semw1).wait()

      one_chunk(2 * j + 1, idx1, rows1, semw1)
      return _

    lax.fori_loop(0, ITH, body, None)
    pltpu.make_async_copy(out_hbm.at[pl.ds(0, CHUNK)], rows0, semw0).wait()
    pltpu.make_async_copy(out_hbm.at[pl.ds(0, CHUNK)], rows1, semw1).wait()

  return gather


def _gather32(table, idx_pad):
  return _make_sc_gather(32)(table, idx_pad)


def _gather16(table, idx_pad):
  return _make_sc_gather(16)(table, idx_pad)


def _pass_a_body(feat, xyz, w1, b1, tab):
  f = _leaky(jnp.dot(feat[...], w1[...],
                     preferred_element_type=jnp.float32) + b1[...])
  tab[:, 0:3] = xyz[...]
  tab[:, 3:19] = f
  tab[:, 19:32] = jnp.zeros((tab.shape[0], 13), jnp.float32)


def _pass_a(feat, xyz3, w1e, b1e):
  return pl.pallas_call(
      _pass_a_body,
      grid=(GRID,),
      in_specs=[
          pl.BlockSpec((BN, 8), lambda i: (i, 0)),
          pl.BlockSpec((BN, 3), lambda i: (i, 0)),
          pl.BlockSpec((8, 16), lambda i: (0, 0)),
          pl.BlockSpec((1, 16), lambda i: (0, 0)),
      ],
      out_specs=pl.BlockSpec((BN, 32), lambda i: (i, 0)),
      out_shape=jax.ShapeDtypeStruct((N, 32), jnp.float32),
  )(feat, xyz3, w1e, b1e)


def _pass_c_body(gth, tab, s4w, d4w, n4w, t4w, f4w, x4w, b1v, a4w, bap1v,
                 bb4w, bbb2v, tab2_o, fx2_o):
  g3 = gth[...]
  g2 = g3.reshape(R4, 128)
  tp = tab[...]
  rel = tp[None] - g3
  rp = (rel * rel).reshape(R4, 128)
  d2 = jnp.dot(rp, s4w[...], preferred_element_type=jnp.float32)
  dv = jnp.sqrt(d2 + 1e-12)
  tt = jnp.dot(tp, t4w[...], preferred_element_type=jnp.float32)
  fx1 = _leaky(
      (jnp.dot(dv, d4w[...], preferred_element_type=jnp.float32)
       + jnp.dot(g2, n4w[...], preferred_element_type=jnp.float32)
       ).reshape(K, B4, 128) + tt[None] + b1v[...][None]).reshape(R4, 128)
  t = (jnp.dot(g2, f4w[...], preferred_element_type=jnp.float32)
       + jnp.dot(fx1, x4w[...], preferred_element_type=jnp.float32))
  t3 = t.reshape(K, B4, 128)
  m = jnp.max(t3, axis=0, keepdims=True)
  e = jnp.exp(t3 - m)
  s = jnp.sum(t3 * e, axis=0) / jnp.sum(e, axis=0)
  tab2_o[...] = _leaky(jnp.dot(s, a4w[...],
                               preferred_element_type=jnp.float32)
                       + bap1v[...])
  fx2 = _leaky(jnp.dot(fx1, bb4w[...], preferred_element_type=jnp.float32)
               + bbb2v[...][None])
  fx2_o[...] = fx2.reshape(K, B4, 128).astype(jnp.bfloat16)


def _pass_c(gth3, tabp, s4w, d4w, n4w, t4w, f4w, x4w, b1v, a4w, bap1v,
            bb4w, bbb2v):
  wspec = pl.BlockSpec((128, 128), lambda i: (0, 0))
  vspec = pl.BlockSpec((1, 128), lambda i: (0, 0))
  return pl.pallas_call(
      _pass_c_body,
      grid=(GRID,),
      in_specs=[
          pl.BlockSpec((K, B4, 128), lambda i: (0, i, 0)),
          pl.BlockSpec((B4, 128), lambda i: (i, 0)),
          wspec, wspec, wspec, wspec, wspec, wspec, vspec, wspec, vspec,
          wspec, vspec,
      ],
      out_specs=[
          pl.BlockSpec((B4, 128), lambda i: (i, 0)),
          pl.BlockSpec((K, B4, 128), lambda i: (0, i, 0)),
      ],
      out_shape=[
          jax.ShapeDtypeStruct((N // 4, 128), jnp.float32),
          jax.ShapeDtypeStruct((K, NP4, 128), jnp.bfloat16),
      ],
  )(gth3, tabp, s4w, d4w, n4w, t4w, f4w, x4w, b1v, a4w, bap1v, bb4w, bbb2v)


def _pass_e_body(gth2, fx2, l24w, x24w, a24w, b24v, agg2_o):
  g2 = gth2[...].reshape(R4, 128)
  f2 = fx2[...].reshape(R4, 128).astype(jnp.float32)
  t = (jnp.dot(g2, l24w[...], preferred_element_type=jnp.float32)
       + jnp.dot(f2, x24w[...], preferred_element_type=jnp.float32))
  t3 = t.reshape(K, B4, 128)
  m = jnp.max(t3, axis=0, keepdims=True)
  e = jnp.exp(t3 - m)
  s = jnp.sum(t3 * e, axis=0) / jnp.sum(e, axis=0)
  agg2_o[...] = _leaky(jnp.dot(s, a24w[...],
                               preferred_element_type=jnp.float32)
                       + b24v[...])


def _pass_e(gth2, fx2, l24w, x24w, a24w, b24v):
  wspec = pl.BlockSpec((128, 128), lambda i: (0, 0))
  vspec = pl.BlockSpec((1, 128), lambda i: (0, 0))
  return pl.pallas_call(
      _pass_e_body,
      grid=(GRID,),
      in_specs=[
          pl.BlockSpec((K, B4, 128), lambda i: (0, i, 0)),
          pl.BlockSpec((K, B4, 128), lambda i: (0, i, 0)),
          wspec, wspec, wspec, vspec,
      ],
      out_specs=pl.BlockSpec((B4, 128), lambda i: (i, 0)),
      out_shape=jax.ShapeDtypeStruct((N // 4, 128), jnp.float32),
  )(gth2, fx2, l24w, x24w, a24w, b24v)


def _pass_f_body(agg2, feat, wm2, bm2, wm3, bm3, out):
  fp2 = jnp.dot(agg2[...], wm2[...],
                preferred_element_type=jnp.float32) + bm2[...]
  scp = jnp.dot(feat[...], wm3[...],
                preferred_element_type=jnp.float32) + bm3[...]
  out[...] = _leaky(fp2 + scp).T


def _pass_f(agg2r, feat, wm2e, bm2e, wm3e, bm3e):
  return pl.pallas_call(
      _pass_f_body,
      grid=(GRID,),
      in_specs=[
          pl.BlockSpec((BN, 32), lambda i: (i, 0)),
          pl.BlockSpec((BN, 8), lambda i: (i, 0)),
          pl.BlockSpec((32, 64), lambda i: (0, 0)),
          pl.BlockSpec((1, 64), lambda i: (0, 0)),
          pl.BlockSpec((8, 64), lambda i: (0, 0)),
          pl.BlockSpec((1, 64), lambda i: (0, 0)),
      ],
      out_specs=pl.BlockSpec((64, BN), lambda i: (0, i)),
      out_shape=jax.ShapeDtypeStruct((64, N), jnp.float32),
  )(agg2r, feat, wm2e, bm2e, wm3e, bm3e)


def _eff(w, b, g, be):
  we = (g[:, None] * w).T
  be_ = (g * b + be).reshape(1, -1)
  return we.astype(jnp.float32), be_.astype(jnp.float32)


def kernel(feature, xyz, neighbour_index,
           w_mlp1, b_mlp1, g_mlp1, be_mlp1,
           w_bb1, b_bb1, g_bb1, be_bb1,
           w_ap1_fc,
           w_ap1_mlp, b_ap1, g_ap1, be_ap1,
           w_bb2, b_bb2, g_bb2, be_bb2,
           w_ap2_fc,
           w_ap2_mlp, b_ap2, g_ap2, be_ap2,
           w_mlp2, b_mlp2, g_mlp2, be_mlp2,
           w_mlp3, b_mlp3, g_mlp3, be_mlp3):
  feat = feature[0, :, :, 0].T
  xyz3 = xyz[0]

  w1e, b1e = _eff(w_mlp1, b_mlp1, g_mlp1, be_mlp1)
  wbb1e, bbb1e = _eff(w_bb1, b_bb1, g_bb1, be_bb1)
  wap1e, bap1e = _eff(w_ap1_mlp, b_ap1, g_ap1, be_ap1)
  wbb2e, bbb2e = _eff(w_bb2, b_bb2, g_bb2, be_bb2)
  wap2e, bap2e = _eff(w_ap2_mlp, b_ap2, g_ap2, be_ap2)
  wm2e, bm2e = _eff(w_mlp2, b_mlp2, g_mlp2, be_mlp2)
  wm3e, bm3e = _eff(w_mlp3, b_mlp3, g_mlp3, be_mlp3)
  wfc1t = w_ap1_fc.T
  wfc2t = w_ap2_fc.T

  idxt = jnp.pad(neighbour_index[0].T, ((0, 0), (0, NP - N)))
  idx_pad = idxt.reshape(NW, IT, CHUNK_J, ROWS_PER_DMA)

  def bd4(w, roff, coff):
    z = jnp.zeros((128, 128), jnp.float32)
    h, wd = w.shape
    for a in range(4):
      z = z.at[32 * a + roff:32 * a + roff + h,
               32 * a + coff:32 * a + coff + wd].set(w)
    return z

  def lane4(v, off=0):
    z = jnp.zeros((32,), jnp.float32).at[off:off + v.shape[0]].set(v)
    return jnp.tile(z, 4)[None]

  s4w = bd4(jnp.ones((3, 1), jnp.float32), 0, 0)
  d4w = bd4(wbb1e[0:1, :], 0, 0)
  n4w = bd4(wbb1e[7:10, :] - wbb1e[1:4, :], 0, 0)
  t4w = bd4(wbb1e[1:4, :] + wbb1e[4:7, :], 0, 0)
  f4w = bd4(wfc1t[0:16, :], 3, 0)
  x4w = bd4(wfc1t[16:32, :], 0, 0)
  b1v = lane4(bbb1e.reshape(-1))
  a4w = bd4(wap1e, 0, 0)
  bap1v = lane4(bap1e.reshape(-1))
  bb4w = bd4(wbb2e, 0, 0)
  bbb2v = lane4(bbb2e.reshape(-1))
  l24w = bd4(wfc2t[0:16, :], 0, 0)
  x24w = bd4(wfc2t[16:32, :], 0, 0)
  a24w = bd4(wap2e, 0, 0)
  b24v = lane4(bap2e.reshape(-1))

  table = _pass_a(feat, xyz3, w1e, b1e)
  tablep = table.reshape(N // 4, 128)
  gth = _gather32(table, idx_pad).reshape(K, NP4, 128)
  tab2p, fx2 = _pass_c(gth, tablep, s4w, d4w, n4w, t4w, f4w, x4w, b1v,
                       a4w, bap1v, bb4w, bbb2v)
  gth2 = _gather32(tab2p.reshape(N, 32), idx_pad).reshape(K, NP4, 128)
  agg2p = _pass_e(gth2, fx2, l24w, x24w, a24w, b24v)
  out = _pass_f(agg2p.reshape(N, 32), feat,
                wm2e, bm2e, wm3e, bm3e)
  return out.reshape(1, 2 * 32, N, 1)


# speedup vs baseline: 7.3317x; 1.0567x over previous
import functools

import jax
import jax.numpy as jnp
from jax import lax
from jax.experimental import pallas as pl
from jax.experimental.pallas import tpu as pltpu
from jax.experimental.pallas import tpu_sc as plsc

N = 100000
K = 16
NP = 102400
NW = 32
CHUNK_J = 10
ROWS_PER_DMA = 128
CHUNK = CHUNK_J * ROWS_PER_DMA
IT = 20
MW = IT * CHUNK
MH = NW * MW
NPH = NP // 2
NPH4 = NPH // 4

BN = 1024
R = BN * K
GRID = -(-N // BN)
B4 = BN // 4
R4 = R // 4
NP4 = NP // 4

_LEAK = 0.2


def _leaky(x):
  return jnp.where(x >= 0, x, _LEAK * x)


@functools.lru_cache(maxsize=None)
def _make_sc_gather(d):
  it = IT
  ith = it // 2
  mesh = plsc.VectorSubcoreMesh(
      core_axis_name="c", subcore_axis_name="s", num_cores=2, num_subcores=16)

  @functools.partial(
      pl.kernel,
      mesh=mesh,
      out_type=jax.ShapeDtypeStruct((MH, d), jnp.float32),
      scratch_types=[
          pltpu.VMEM((CHUNK_J, ROWS_PER_DMA), jnp.int32),
          pltpu.VMEM((CHUNK_J, ROWS_PER_DMA), jnp.int32),
          pltpu.VMEM((CHUNK, d), jnp.float32),
          pltpu.VMEM((CHUNK, d), jnp.float32),
          pltpu.SemaphoreType.DMA,
          pltpu.SemaphoreType.DMA,
          pltpu.SemaphoreType.DMA,
      ],
      compiler_params=pltpu.CompilerParams(use_tc_tiling_on_sc=False),
  )
  def gather(table_hbm, idx_hbm, out_hbm, idx0, idx1, rows0, rows1,
             semg, semw0, semw1):
    wid = lax.axis_index("s") * 2 + lax.axis_index("c")

    def one_chunk(chunk, idx_v, rows_v, semw):
      base = (wid * it + chunk) * CHUNK
      pltpu.sync_copy(idx_hbm.at[wid, chunk], idx_v)
      copies = []
      for j in range(CHUNK_J):
        copies.append(
            pltpu.async_copy(
                table_hbm.at[idx_v.at[j]],
                rows_v.at[pl.ds(j * ROWS_PER_DMA, ROWS_PER_DMA)],
                semg,
            )
        )
      for c in copies:
        c.wait()
      pltpu.async_copy(rows_v, out_hbm.at[pl.ds(base, CHUNK)], semw)

    def body(j, _):
      @pl.when(j >= 1)
      def _drain0():
        pltpu.make_async_copy(
            out_hbm.at[pl.ds(0, CHUNK)], rows0, semw0).wait()

      one_chunk(2 * j, idx0, rows0, semw0)

      @pl.when(j >= 1)
      def _drain1():
        pltpu.make_async_copy(
            out_hbm.at[pl.ds(0, CHUNK)], rows1, semw1).wait()

      one_chunk(2 * j + 1, idx1, rows1, semw1)
      return _

    lax.fori_loop(0, ith, body, None)
    pltpu.make_async_copy(out_hbm.at[pl.ds(0, CHUNK)], rows0, semw0).wait()
    pltpu.make_async_copy(out_hbm.at[pl.ds(0, CHUNK)], rows1, semw1).wait()

  return gather


def _gather32(table, idx_pad):
  return _make_sc_gather(32)(table, idx_pad)


def _gather16(table, idx_pad):
  return _make_sc_gather(16)(table, idx_pad)


def _pass_a_body(feat, xyz, w1, b1, tab):
  f = _leaky(jnp.dot(feat[...], w1[...],
                     preferred_element_type=jnp.float32) + b1[...])
  tab[:, 0:3] = xyz[...]
  tab[:, 3:19] = f
  tab[:, 19:32] = jnp.zeros((tab.shape[0], 13), jnp.float32)


def _pass_a(feat, xyz3, w1e, b1e):
  return pl.pallas_call(
      _pass_a_body,
      grid=(GRID,),
      in_specs=[
          pl.BlockSpec((BN, 8), lambda i: (i, 0)),
          pl.BlockSpec((BN, 3), lambda i: (i, 0)),
          pl.BlockSpec((8, 16), lambda i: (0, 0)),
          pl.BlockSpec((1, 16), lambda i: (0, 0)),
      ],
      out_specs=pl.BlockSpec((BN, 32), lambda i: (i, 0)),
      out_shape=jax.ShapeDtypeStruct((N, 32), jnp.float32),
  )(feat, xyz3, w1e, b1e)


def _pass_c_body(gth, tab, s4w, d4w, n4w, t4w, f4w, x4w, b1v, a4w, bap1v,
                 bb4w, bbb2v, tab2_o, fx2_o):
  g3 = gth[...]
  g2 = g3.reshape(R4, 128)
  tp = tab[...]
  rel = tp[None] - g3
  rp = (rel * rel).reshape(R4, 128)
  d2 = jnp.dot(rp, s4w[...], preferred_element_type=jnp.float32)
  dv = jnp.sqrt(d2 + 1e-12)
  tt = jnp.dot(tp, t4w[...], preferred_element_type=jnp.float32)
  fx1 = _leaky(
      (jnp.dot(dv, d4w[...], preferred_element_type=jnp.float32)
       + jnp.dot(g2, n4w[...], preferred_element_type=jnp.float32)
       ).reshape(K, B4, 128) + tt[None] + b1v[...][None]).reshape(R4, 128)
  t = (jnp.dot(g2, f4w[...], preferred_element_type=jnp.float32)
       + jnp.dot(fx1, x4w[...], preferred_element_type=jnp.float32))
  t3 = t.reshape(K, B4, 128)
  m = jnp.max(t3, axis=0, keepdims=True)
  e = jnp.exp(t3 - m)
  s = jnp.sum(t3 * e, axis=0) / jnp.sum(e, axis=0)
  tab2_o[...] = _leaky(jnp.dot(s, a4w[...],
                               preferred_element_type=jnp.float32)
                       + bap1v[...])
  fx2 = _leaky(jnp.dot(fx1, bb4w[...], preferred_element_type=jnp.float32)
               + bbb2v[...][None])
  fx2_o[...] = fx2.reshape(K, B4, 128).astype(jnp.bfloat16)


def _pass_c(h, npts, gth3, tabp, s4w, d4w, n4w, t4w, f4w, x4w, b1v, a4w,
            bap1v, bb4w, bbb2v):
  grid = -(-npts // BN)
  off = h * (NPH // BN)
  wspec = pl.BlockSpec((128, 128), lambda i: (0, 0))
  vspec = pl.BlockSpec((1, 128), lambda i: (0, 0))
  return pl.pallas_call(
      _pass_c_body,
      grid=(grid,),
      in_specs=[
          pl.BlockSpec((K, B4, 128), lambda i: (0, i, 0)),
          pl.BlockSpec((B4, 128), lambda i: (i + off, 0)),
          wspec, wspec, wspec, wspec, wspec, wspec, vspec, wspec, vspec,
          wspec, vspec,
      ],
      out_specs=[
          pl.BlockSpec((B4, 128), lambda i: (i, 0)),
          pl.BlockSpec((K, B4, 128), lambda i: (0, i, 0)),
      ],
      out_shape=[
          jax.ShapeDtypeStruct((-(-npts // 4) + 0, 128), jnp.float32),
          jax.ShapeDtypeStruct((K, NPH4, 128), jnp.bfloat16),
      ],
  )(gth3, tabp, s4w, d4w, n4w, t4w, f4w, x4w, b1v, a4w, bap1v, bb4w, bbb2v)


def _pass_e_body(gth2, fx2, l24w, x24w, a24w, b24v, agg2_o):
  g2 = gth2[...].reshape(R4, 128)
  f2 = fx2[...].reshape(R4, 128).astype(jnp.float32)
  t = (jnp.dot(g2, l24w[...], preferred_element_type=jnp.float32)
       + jnp.dot(f2, x24w[...], preferred_element_type=jnp.float32))
  t3 = t.reshape(K, B4, 128)
  m = jnp.max(t3, axis=0, keepdims=True)
  e = jnp.exp(t3 - m)
  s = jnp.sum(t3 * e, axis=0) / jnp.sum(e, axis=0)
  agg2_o[...] = _leaky(jnp.dot(s, a24w[...],
                               preferred_element_type=jnp.float32)
                       + b24v[...])


def _pass_e(npts, gth2, fx2, l24w, x24w, a24w, b24v):
  grid = -(-npts // BN)
  wspec = pl.BlockSpec((128, 128), lambda i: (0, 0))
  vspec = pl.BlockSpec((1, 128), lambda i: (0, 0))
  return pl.pallas_call(
      _pass_e_body,
      grid=(grid,),
      in_specs=[
          pl.BlockSpec((K, B4, 128), lambda i: (0, i, 0)),
          pl.BlockSpec((K, B4, 128), lambda i: (0, i, 0)),
          wspec, wspec, wspec, vspec,
      ],
      out_specs=pl.BlockSpec((B4, 128), lambda i: (i, 0)),
      out_shape=jax.ShapeDtypeStruct((-(-npts // 4), 128), jnp.float32),
  )(gth2, fx2, l24w, x24w, a24w, b24v)


def _pass_f_body(agg2, feat, wm2, bm2, wm3, bm3, out):
  fp2 = jnp.dot(agg2[...], wm2[...],
                preferred_element_type=jnp.float32) + bm2[...]
  scp = jnp.dot(feat[...], wm3[...],
                preferred_element_type=jnp.float32) + bm3[...]
  out[...] = _leaky(fp2 + scp).T


def _pass_f(agg2r, feat, wm2e, bm2e, wm3e, bm3e):
  return pl.pallas_call(
      _pass_f_body,
      grid=(GRID,),
      in_specs=[
          pl.BlockSpec((BN, 32), lambda i: (i, 0)),
          pl.BlockSpec((BN, 8), lambda i: (i, 0)),
          pl.BlockSpec((32, 64), lambda i: (0, 0)),
          pl.BlockSpec((1, 64), lambda i: (0, 0)),
          pl.BlockSpec((8, 64), lambda i: (0, 0)),
          pl.BlockSpec((1, 64), lambda i: (0, 0)),
      ],
      out_specs=pl.BlockSpec((64, BN), lambda i: (0, i)),
      out_shape=jax.ShapeDtypeStruct((64, N), jnp.float32),
  )(agg2r, feat, wm2e, bm2e, wm3e, bm3e)


def _eff(w, b, g, be):
  we = (g[:, None] * w).T
  be_ = (g * b + be).reshape(1, -1)
  return we.astype(jnp.float32), be_.astype(jnp.float32)


def kernel(feature, xyz, neighbour_index,
           w_mlp1, b_mlp1, g_mlp1, be_mlp1,
           w_bb1, b_bb1, g_bb1, be_bb1,
           w_ap1_fc,
           w_ap1_mlp, b_ap1, g_ap1, be_ap1,
           w_bb2, b_bb2, g_bb2, be_bb2,
           w_ap2_fc,
           w_ap2_mlp, b_ap2, g_ap2, be_ap2,
           w_mlp2, b_mlp2, g_mlp2, be_mlp2,
           w_mlp3, b_mlp3, g_mlp3, be_mlp3):
  feat = feature[0, :, :, 0].T
  xyz3 = xyz[0]

  w1e, b1e = _eff(w_mlp1, b_mlp1, g_mlp1, be_mlp1)
  wbb1e, bbb1e = _eff(w_bb1, b_bb1, g_bb1, be_bb1)
  wap1e, bap1e = _eff(w_ap1_mlp, b_ap1, g_ap1, be_ap1)
  wbb2e, bbb2e = _eff(w_bb2, b_bb2, g_bb2, be_bb2)
  wap2e, bap2e = _eff(w_ap2_mlp, b_ap2, g_ap2, be_ap2)
  wm2e, bm2e = _eff(w_mlp2, b_mlp2, g_mlp2, be_mlp2)
  wm3e, bm3e = _eff(w_mlp3, b_mlp3, g_mlp3, be_mlp3)
  wfc1t = w_ap1_fc.T
  wfc2t = w_ap2_fc.T

  idxt = jnp.pad(neighbour_index[0].T, ((0, 0), (0, NP - N)))
  idx_h = [
      idxt[:, :NPH].reshape(NW, IT, CHUNK_J, ROWS_PER_DMA),
      idxt[:, NPH:].reshape(NW, IT, CHUNK_J, ROWS_PER_DMA),
  ]
  np_h = [NPH, N - NPH]

  def bd4(w, roff, coff):
    z = jnp.zeros((128, 128), jnp.float32)
    h, wd = w.shape
    for a in range(4):
      z = z.at[32 * a + roff:32 * a + roff + h,
               32 * a + coff:32 * a + coff + wd].set(w)
    return z

  def lane4(v, off=0):
    z = jnp.zeros((32,), jnp.float32).at[off:off + v.shape[0]].set(v)
    return jnp.tile(z, 4)[None]

  s4w = bd4(jnp.ones((3, 1), jnp.float32), 0, 0)
  d4w = bd4(wbb1e[0:1, :], 0, 0)
  n4w = bd4(wbb1e[7:10, :] - wbb1e[1:4, :], 0, 0)
  t4w = bd4(wbb1e[1:4, :] + wbb1e[4:7, :], 0, 0)
  f4w = bd4(wfc1t[0:16, :], 3, 0)
  x4w = bd4(wfc1t[16:32, :], 0, 0)
  b1v = lane4(bbb1e.reshape(-1))
  a4w = bd4(wap1e, 0, 0)
  bap1v = lane4(bap1e.reshape(-1))
  bb4w = bd4(wbb2e, 0, 0)
  bbb2v = lane4(bbb2e.reshape(-1))
  l24w = bd4(wfc2t[0:16, :], 0, 0)
  x24w = bd4(wfc2t[16:32, :], 0, 0)
  a24w = bd4(wap2e, 0, 0)
  b24v = lane4(bap2e.reshape(-1))

  table = _pass_a(feat, xyz3, w1e, b1e)
  tablep = table.reshape(N // 4, 128)
  gth_h = [_gather32(table, idx_h[h]).reshape(K, NPH4, 128)
           for h in (0, 1)]
  tab2_h = []
  fx2_h = []
  for h in (0, 1):
    t2, f2 = _pass_c(h, np_h[h], gth_h[h], tablep, s4w, d4w, n4w, t4w,
                     f4w, x4w, b1v, a4w, bap1v, bb4w, bbb2v)
    tab2_h.append(t2)
    fx2_h.append(f2)
  tab2 = jnp.concatenate(tab2_h, axis=0).reshape(N, 32)
  gth2_h = [_gather32(tab2, idx_h[h]).reshape(K, NPH4, 128)
            for h in (0, 1)]
  agg2_h = [_pass_e(np_h[h], gth2_h[h], fx2_h[h], l24w, x24w, a24w, b24v)
            for h in (0, 1)]
  agg2 = jnp.concatenate(agg2_h, axis=0).reshape(N, 32)
  out = _pass_f(agg2, feat, wm2e, bm2e, wm3e, bm3e)
  return out.reshape(1, 2 * 32, N, 1)
